# baseline (speedup 1.0000x reference)
import jax
import jax.numpy as jnp
from jax import lax
from jax.experimental import pallas as pl


def _moe_body(x_ref, Wg1_ref, bg1_ref, Wg2_ref, bg2_ref, We_ref, be3_ref,
              out_ref):
    S = x_ref.shape[0]
    C = x_ref.shape[1]
    E = We_ref.shape[0]
    for s in range(S):
        xs = x_ref[s]
        pooled = jnp.mean(xs, axis=1, keepdims=True)
        h = lax.dot_general(Wg1_ref[...], pooled, (((1,), (0,)), ((), ())),
                            preferred_element_type=jnp.float32)
        h = jnp.maximum(h + bg1_ref[...], 0.0)
        logits = lax.dot_general(Wg2_ref[...], h, (((1,), (0,)), ((), ())),
                                 preferred_element_type=jnp.float32)
        logits = logits + bg2_ref[...]
        lmax = jnp.max(logits)
        expv = jnp.exp(logits - lmax)
        probs = expv / jnp.sum(expv)
        eidx = lax.broadcasted_iota(jnp.int32, probs.shape, 0)
        m1 = jnp.max(probs)
        i1 = jnp.min(jnp.where(probs == m1, eidx, E))
        probs2 = jnp.where(eidx == i1, -1.0, probs)
        m2 = jnp.max(probs2)
        i2 = jnp.min(jnp.where(probs2 == m2, eidx, E))
        inv = 1.0 / (m1 + m2 + 1e-8)
        v1 = m1 * inv
        v2 = m2 * inv
        S1 = We_ref[pl.ds(i1, 1), :, :][0]
        S2 = We_ref[pl.ds(i2, 1), :, :][0]
        Wc = v1 * S1 + v2 * S2
        bc = (v1 * be3_ref[pl.ds(i1, 1), :, :][0]
              + v2 * be3_ref[pl.ds(i2, 1), :, :][0])
        y = lax.dot_general(Wc, xs, (((1,), (0,)), ((), ())),
                            preferred_element_type=jnp.float32)
        out_ref[s] = y + bc + xs


def kernel(x, Wg1, bg1, Wg2, bg2, We, be):
    B, C, H, W = x.shape
    E, GH = Wg2.shape
    HW = H * W
    G = 4
    S = B // G
    x3 = x.reshape(B, C, HW)
    out = pl.pallas_call(
        _moe_body,
        grid=(G,),
        in_specs=[
            pl.BlockSpec((S, C, HW), lambda b: (b, 0, 0)),
            pl.BlockSpec((GH, C), lambda b: (0, 0)),
            pl.BlockSpec((GH, 1), lambda b: (0, 0)),
            pl.BlockSpec((E, GH), lambda b: (0, 0)),
            pl.BlockSpec((E, 1), lambda b: (0, 0)),
            pl.BlockSpec((E, C, C), lambda b: (0, 0, 0)),
            pl.BlockSpec((E, C, 1), lambda b: (0, 0, 0)),
        ],
        out_specs=pl.BlockSpec((S, C, HW), lambda b: (b, 0, 0)),
        out_shape=jax.ShapeDtypeStruct((B, C, HW), jnp.float32),
    )(x3, Wg1, bg1.reshape(GH, 1), Wg2, bg2.reshape(E, 1), We, be[:, :, None])
    return out.reshape(B, C, H, W)
